# baseline (speedup 1.0000x reference)
import jax
import jax.numpy as jnp
from jax import lax
from jax.experimental import pallas as pl
from jax.experimental.pallas import tpu as pltpu
from jax.experimental.pallas import tpu_sc as plsc

N = 10000
E = 320000
DF = 128
DE = 4
DEP = 16
DH = 300
DHP = 384
G = 128

NC, NS = 2, 16
NW = NC * NS
CHUNK = 64
CPW = 160
BLK = 16
EPAD = NW * CPW * CHUNK
NP = 10240
RPT = NP // NS
ZR = 64
EB = 1024


def _sc_body(x_hbm, src_hbm, dst_hbm, e_hbm, outx_hbm,
             aggx_sh, sblk, dblk, xbuf0, xbuf1, ebuf0, ebuf1,
             xsem0, xsem1, esem0, esem1):
    cid = lax.axis_index("c")
    sid = lax.axis_index("s")
    wid = cid * NS + sid
    xbufs, ebufs = (xbuf0, xbuf1), (ebuf0, ebuf1)
    xsems, esems = (xsem0, xsem1), (esem0, esem1)

    zeros16 = jnp.zeros((16,), jnp.float32)

    @pl.loop(0, ZR)
    def _zero(i):
        for j in range(DF // 16):
            xbuf0[i, pl.ds(j * 16, 16)] = zeros16

    def _ramp(base):
        for j in range(CHUNK // 16):
            sblk[0, 0, pl.ds(j * 16, 16)] = lax.iota(jnp.int32, 16) + (base + j * 16)

    for k in range(RPT // ZR):
        r0 = sid * RPT + k * ZR
        _ramp(r0)
        pltpu.sync_copy(xbuf0, aggx_sh.at[sblk.at[0, 0]])
    plsc.subcore_barrier()

    def _load_blk(c):
        row = wid * CPW + c
        pltpu.sync_copy(src_hbm.at[pl.ds(row, BLK)], sblk)
        pltpu.sync_copy(dst_hbm.at[pl.ds(row, BLK)], dblk)

    def _fire(c, b):
        base = (wid * CPW + c) * CHUNK
        j = jnp.bitwise_and(c, BLK - 1)
        pltpu.async_copy(x_hbm.at[sblk.at[j, 0]], xbufs[b], xsems[b])
        pltpu.async_copy(e_hbm.at[pl.ds(base, CHUNK), :], ebufs[b], esems[b])

    def _drain_and_add(c, b):
        base = (wid * CPW + c) * CHUNK
        j = jnp.bitwise_and(c, BLK - 1)
        pltpu.make_async_copy(x_hbm.at[sblk.at[j, 0]], xbufs[b], xsems[b]).wait()
        pltpu.make_async_copy(
            e_hbm.at[pl.ds(base, CHUNK), :], ebufs[b], esems[b]).wait()
        pltpu.sync_copy(xbufs[b], aggx_sh.at[dblk.at[j, 0]], add=True)
        pltpu.sync_copy(ebufs[b], aggx_sh.at[dblk.at[j, 0]], add=True)

    _load_blk(0)
    _fire(0, 0)

    @pl.loop(0, CPW // 2)
    def _edges(g):
        for b in range(2):
            c = g * 2 + b
            nc = c + 1
            at_blk_end = jnp.bitwise_and(nc, BLK - 1) == 0

            @pl.when(jnp.logical_and(nc < CPW, jnp.logical_not(at_blk_end)))
            def _():
                _fire(nc, 1 - b)

            _drain_and_add(c, b)

            @pl.when(jnp.logical_and(nc < CPW, at_blk_end))
            def _():
                _load_blk(nc)
                _fire(nc, 1 - b)

    plsc.subcore_barrier()

    for k in range(RPT // ZR):
        r0 = sid * RPT + k * ZR
        o0 = cid * NP + r0
        _ramp(r0)
        pltpu.sync_copy(aggx_sh.at[sblk.at[0, 0]], xbuf0)
        pltpu.sync_copy(xbuf0, outx_hbm.at[pl.ds(o0, ZR), :])


def _sc_agg(x_pad, src_pad, dst_pad, e_rows):
    return pl.kernel(
        _sc_body,
        out_type=jax.ShapeDtypeStruct((NC * NP, DF), jnp.float32),
        mesh=plsc.VectorSubcoreMesh(core_axis_name="c", subcore_axis_name="s"),
        scratch_types=[
            pltpu.VMEM_SHARED((NP, DF), jnp.float32),
            pltpu.VMEM((BLK, 1, CHUNK), jnp.int32),
            pltpu.VMEM((BLK, 1, CHUNK), jnp.int32),
            pltpu.VMEM((CHUNK, DF), jnp.float32),
            pltpu.VMEM((CHUNK, DF), jnp.float32),
            pltpu.VMEM((CHUNK, DF), jnp.float32),
            pltpu.VMEM((CHUNK, DF), jnp.float32),
            pltpu.SemaphoreType.DMA,
            pltpu.SemaphoreType.DMA,
            pltpu.SemaphoreType.DMA,
            pltpu.SemaphoreType.DMA,
        ],
    )(x_pad, src_pad, dst_pad, e_rows)


def _edge_mm_body(ea_ref, we_ref, out_ref):
    out_ref[...] = jnp.dot(ea_ref[...], we_ref[...],
                           preferred_element_type=jnp.float32)


_edge_mm = pl.pallas_call(
    _edge_mm_body,
    grid=(EPAD // EB,),
    in_specs=[
        pl.BlockSpec((EB, DEP), lambda i: (i, 0)),
        pl.BlockSpec((DEP, DF), lambda i: (0, 0)),
    ],
    out_specs=pl.BlockSpec((EB, DF), lambda i: (i, 0)),
    out_shape=jax.ShapeDtypeStruct((EPAD, DF), jnp.float32),
)


def _tc_body(paggx, x, batch, W_gnn, b_gnn, W_imp, b_imp,
             W_p1, b_p1, W_p2, b_p2, out_ref):
    f32 = jnp.float32
    t = paggx[0:N, :] + paggx[NP:NP + N, :] + x[...]

    imp_pre = jnp.dot(t, W_imp[...], preferred_element_type=f32) + b_imp[0, 0]
    imp = 1.0 / (1.0 + jnp.exp(-imp_pre))
    impT_pre = lax.dot_general(W_imp[...], t, (((0,), (1,)), ((), ())),
                               preferred_element_type=f32) + b_imp[0, 0]
    impT = 1.0 / (1.0 + jnp.exp(-impT_pre))

    h = jnp.maximum(jnp.dot(t, W_gnn[...], preferred_element_type=f32)
                    + b_gnn[...], 0.0)

    gid = lax.broadcasted_iota(jnp.int32, (G, N), 0)
    oh = jnp.broadcast_to(batch[...], (G, N)) == gid
    ohf = oh.astype(f32)

    impb = jnp.broadcast_to(impT[0:1, :], (G, N))
    m10 = jnp.max(jnp.where(oh, impb, -1.0), axis=1, keepdims=True) * 10.0
    node_m = lax.dot_general(ohf, m10, (((0,), (0,)), ((), ())),
                             preferred_element_type=f32)
    wgt = imp[:, 0:1] / node_m + 0.9

    hw = h * jnp.broadcast_to(wgt, (N, DHP))
    sums = jnp.dot(ohf, hw, preferred_element_type=f32)
    counts = jnp.sum(ohf, axis=1, keepdims=True)
    pooled = sums / jnp.maximum(counts, 1.0)

    hid = jnp.maximum(jnp.dot(pooled, W_p1[...], preferred_element_type=f32)
                      + b_p1[...], 0.0)
    out_ref[...] = jnp.dot(hid, W_p2[...], preferred_element_type=f32) + b_p2[...]


_tc_call = pl.pallas_call(
    _tc_body,
    out_shape=jax.ShapeDtypeStruct((G, DHP), jnp.float32),
)


def kernel(x, edge_index, edge_attr, batch, W_edge, W_gnn, b_gnn, W_imp, b_imp,
           W_p1, b_p1, W_p2, b_p2):
    src = edge_index[0]
    dst = edge_index[1]

    pad = EPAD - E
    x_pad = jnp.concatenate([x, jnp.zeros((8, DF), jnp.float32)], axis=0)
    src_pad = jnp.concatenate([src, jnp.full((pad,), N, jnp.int32)])
    dst_pad = jnp.concatenate([dst, jnp.zeros((pad,), jnp.int32)])
    src3d = src_pad.reshape(EPAD // CHUNK, 1, CHUNK)
    dst3d = dst_pad.reshape(EPAD // CHUNK, 1, CHUNK)
    ea_pad = jnp.pad(edge_attr, ((0, pad), (0, DEP - DE)))
    W_edge_p = jnp.pad(W_edge, ((0, DEP - DE), (0, 0)))

    e_rows = _edge_mm(ea_pad, W_edge_p)
    paggx = _sc_agg(x_pad, src3d, dst3d, e_rows)

    W_gnn_p = jnp.pad(W_gnn, ((0, 0), (0, DHP - DH)))
    b_gnn_p = jnp.pad(b_gnn, (0, DHP - DH)).reshape(1, DHP)
    W_imp_p = jnp.broadcast_to(W_imp, (DF, 8))
    b_imp_p = b_imp.reshape(1, 1)
    W_p1_p = jnp.pad(W_p1, ((0, DHP - DH), (0, DHP - DH)))
    b_p1_p = jnp.pad(b_p1, (0, DHP - DH)).reshape(1, DHP)
    W_p2_p = jnp.pad(W_p2, ((0, DHP - DH), (0, DHP - DH)))
    b_p2_p = jnp.pad(b_p2, (0, DHP - DH)).reshape(1, DHP)
    batch2 = batch.reshape(1, N)

    z = _tc_call(paggx, x, batch2, W_gnn_p, b_gnn_p,
                 W_imp_p, b_imp_p, W_p1_p, b_p1_p, W_p2_p, b_p2_p)
    return z[:, :DH]


# speedup vs baseline: 1.1701x; 1.1701x over previous
import jax
import jax.numpy as jnp
from jax import lax
from jax.experimental import pallas as pl
from jax.experimental.pallas import tpu as pltpu
from jax.experimental.pallas import tpu_sc as plsc

N = 10000
E = 320000
DF = 128
DE = 4
DEP = 16
DH = 300
DHP = 384
G = 128

NC, NS = 2, 16
NW = NC * NS
CHUNK = 64
CPW = 158
EPAD = NW * CPW * CHUNK
NP = 10240
RPT = NP // NS
ZR = 64
EB = 1024


def _sc_body(x_hbm, sd_hbm, e_hbm, outx_hbm,
             aggx_sh, sd0, sd1, xbuf0, xbuf1, ebuf0, ebuf1,
             xsem0, xsem1, esem0, esem1, sxsem0, sxsem1, sesem0, sesem1):
    cid = lax.axis_index("c")
    sid = lax.axis_index("s")
    wid = cid * NS + sid
    sds = (sd0, sd1)
    xbufs, ebufs = (xbuf0, xbuf1), (ebuf0, ebuf1)
    xsems, esems = (xsem0, xsem1), (esem0, esem1)
    sxsems, sesems = (sxsem0, sxsem1), (sesem0, sesem1)

    zeros16 = jnp.zeros((16,), jnp.float32)

    @pl.loop(0, ZR)
    def _zero(i):
        for j in range(DF // 16):
            sl = pl.ds(j * 16, 16)
            xbuf0[i, sl] = zeros16
            xbuf1[i, sl] = zeros16
            ebuf0[i, sl] = zeros16
            ebuf1[i, sl] = zeros16

    def _ramp(ref, r, base):
        for j in range(CHUNK // 16):
            ref[r, 0, pl.ds(j * 16, 16)] = lax.iota(jnp.int32, 16) + (base + j * 16)

    for k in range(RPT // ZR):
        r0 = sid * RPT + k * ZR
        _ramp(sd0, 0, r0)
        pltpu.sync_copy(xbuf0, aggx_sh.at[sd0.at[0, 0]])
    plsc.subcore_barrier()

    def _scat_x(b):
        return pltpu.make_async_copy(xbufs[b], aggx_sh.at[sds[b].at[1, 0]],
                                     sxsems[b])

    def _scat_e(b):
        return pltpu.make_async_copy(ebufs[b], aggx_sh.at[sds[b].at[1, 0]],
                                     sesems[b])

    for b in range(2):
        _ramp(sds[b], 1, 0)
        _scat_x(b).start(add=True)
        _scat_e(b).start(add=True)

    def _fire(c, b):
        base = (wid * CPW + c) * CHUNK
        _scat_x(b).wait()
        _scat_e(b).wait()
        pltpu.sync_copy(sd_hbm.at[wid * CPW + c], sds[b])
        pltpu.async_copy(x_hbm.at[sds[b].at[0, 0]], xbufs[b], xsems[b])
        pltpu.async_copy(e_hbm.at[pl.ds(base, CHUNK), :], ebufs[b], esems[b])

    def _drain_and_add(c, b):
        base = (wid * CPW + c) * CHUNK
        pltpu.make_async_copy(x_hbm.at[sds[b].at[0, 0]], xbufs[b],
                              xsems[b]).wait()
        pltpu.make_async_copy(
            e_hbm.at[pl.ds(base, CHUNK), :], ebufs[b], esems[b]).wait()
        _scat_x(b).start(add=True)
        _scat_e(b).start(add=True)

    _fire(0, 0)

    @pl.loop(0, CPW // 2)
    def _edges(g):
        for b in range(2):
            c = g * 2 + b

            @pl.when(c + 1 < CPW)
            def _():
                _fire(c + 1, 1 - b)

            _drain_and_add(c, b)

    _scat_x(0).wait()
    _scat_e(0).wait()
    _scat_x(1).wait()
    _scat_e(1).wait()
    plsc.subcore_barrier()

    for k in range(RPT // ZR):
        r0 = sid * RPT + k * ZR
        o0 = cid * NP + r0
        _ramp(sd0, 0, r0)
        pltpu.sync_copy(aggx_sh.at[sd0.at[0, 0]], xbuf0)
        pltpu.sync_copy(xbuf0, outx_hbm.at[pl.ds(o0, ZR), :])


def _sc_agg(x_pad, sd4, e_rows):
    return pl.kernel(
        _sc_body,
        out_type=jax.ShapeDtypeStruct((NC * NP, DF), jnp.float32),
        mesh=plsc.VectorSubcoreMesh(core_axis_name="c", subcore_axis_name="s"),
        scratch_types=[
            pltpu.VMEM_SHARED((NP, DF), jnp.float32),
            pltpu.VMEM((2, 1, CHUNK), jnp.int32),
            pltpu.VMEM((2, 1, CHUNK), jnp.int32),
            pltpu.VMEM((CHUNK, DF), jnp.float32),
            pltpu.VMEM((CHUNK, DF), jnp.float32),
            pltpu.VMEM((CHUNK, DF), jnp.float32),
            pltpu.VMEM((CHUNK, DF), jnp.float32),
            pltpu.SemaphoreType.DMA,
            pltpu.SemaphoreType.DMA,
            pltpu.SemaphoreType.DMA,
            pltpu.SemaphoreType.DMA,
            pltpu.SemaphoreType.DMA,
            pltpu.SemaphoreType.DMA,
            pltpu.SemaphoreType.DMA,
            pltpu.SemaphoreType.DMA,
        ],
    )(x_pad, sd4, e_rows)


def _edge_mm_body(ea_ref, we_ref, out_ref):
    out_ref[...] = jnp.dot(ea_ref[...], we_ref[...],
                           preferred_element_type=jnp.float32)


_edge_mm = pl.pallas_call(
    _edge_mm_body,
    grid=(EPAD // EB,),
    in_specs=[
        pl.BlockSpec((EB, DEP), lambda i: (i, 0)),
        pl.BlockSpec((DEP, DF), lambda i: (0, 0)),
    ],
    out_specs=pl.BlockSpec((EB, DF), lambda i: (i, 0)),
    out_shape=jax.ShapeDtypeStruct((EPAD, DF), jnp.float32),
)


def _tc_body(paggx, x, batch, W_gnn, b_gnn, W_imp, b_imp,
             W_p1, b_p1, W_p2, b_p2, out_ref):
    f32 = jnp.float32
    t = paggx[0:N, :] + paggx[NP:NP + N, :] + x[...]

    imp_pre = jnp.dot(t, W_imp[...], preferred_element_type=f32) + b_imp[0, 0]
    imp = 1.0 / (1.0 + jnp.exp(-imp_pre))
    impT_pre = lax.dot_general(W_imp[...], t, (((0,), (1,)), ((), ())),
                               preferred_element_type=f32) + b_imp[0, 0]
    impT = 1.0 / (1.0 + jnp.exp(-impT_pre))

    h = jnp.maximum(jnp.dot(t, W_gnn[...], preferred_element_type=f32)
                    + b_gnn[...], 0.0)

    gid = lax.broadcasted_iota(jnp.int32, (G, N), 0)
    oh = jnp.broadcast_to(batch[...], (G, N)) == gid
    ohf = oh.astype(f32)

    impb = jnp.broadcast_to(impT[0:1, :], (G, N))
    m10 = jnp.max(jnp.where(oh, impb, -1.0), axis=1, keepdims=True) * 10.0
    node_m = lax.dot_general(ohf, m10, (((0,), (0,)), ((), ())),
                             preferred_element_type=f32)
    wgt = imp[:, 0:1] / node_m + 0.9

    hw = h * jnp.broadcast_to(wgt, (N, DHP))
    sums = jnp.dot(ohf, hw, preferred_element_type=f32)
    counts = jnp.sum(ohf, axis=1, keepdims=True)
    pooled = sums / jnp.maximum(counts, 1.0)

    hid = jnp.maximum(jnp.dot(pooled, W_p1[...], preferred_element_type=f32)
                      + b_p1[...], 0.0)
    out_ref[...] = jnp.dot(hid, W_p2[...], preferred_element_type=f32) + b_p2[...]


_tc_call = pl.pallas_call(
    _tc_body,
    out_shape=jax.ShapeDtypeStruct((G, DHP), jnp.float32),
)


def kernel(x, edge_index, edge_attr, batch, W_edge, W_gnn, b_gnn, W_imp, b_imp,
           W_p1, b_p1, W_p2, b_p2):
    src = edge_index[0]
    dst = edge_index[1]

    pad = EPAD - E
    x_pad = jnp.concatenate([x, jnp.zeros((8, DF), jnp.float32)], axis=0)
    src_pad = jnp.concatenate([src, jnp.full((pad,), N, jnp.int32)])
    dst_pad = jnp.concatenate([dst, jnp.zeros((pad,), jnp.int32)])
    sd4 = jnp.stack([src_pad.reshape(EPAD // CHUNK, CHUNK),
                     dst_pad.reshape(EPAD // CHUNK, CHUNK)],
                    axis=1).reshape(EPAD // CHUNK, 2, 1, CHUNK)
    ea_pad = jnp.pad(edge_attr, ((0, pad), (0, DEP - DE)))
    W_edge_p = jnp.pad(W_edge, ((0, DEP - DE), (0, 0)))

    e_rows = _edge_mm(ea_pad, W_edge_p)
    paggx = _sc_agg(x_pad, sd4, e_rows)

    W_gnn_p = jnp.pad(W_gnn, ((0, 0), (0, DHP - DH)))
    b_gnn_p = jnp.pad(b_gnn, (0, DHP - DH)).reshape(1, DHP)
    W_imp_p = jnp.broadcast_to(W_imp, (DF, 8))
    b_imp_p = b_imp.reshape(1, 1)
    W_p1_p = jnp.pad(W_p1, ((0, DHP - DH), (0, DHP - DH)))
    b_p1_p = jnp.pad(b_p1, (0, DHP - DH)).reshape(1, DHP)
    W_p2_p = jnp.pad(W_p2, ((0, DHP - DH), (0, DHP - DH)))
    b_p2_p = jnp.pad(b_p2, (0, DHP - DH)).reshape(1, DHP)
    batch2 = batch.reshape(1, N)

    z = _tc_call(paggx, x, batch2, W_gnn_p, b_gnn_p,
                 W_imp_p, b_imp_p, W_p1_p, b_p1_p, W_p2_p, b_p2_p)
    return z[:, :DH]


# speedup vs baseline: 1.2060x; 1.0306x over previous
import jax
import jax.numpy as jnp
from jax import lax
from jax.experimental import pallas as pl
from jax.experimental.pallas import tpu as pltpu
from jax.experimental.pallas import tpu_sc as plsc

N = 10000
E = 320000
DF = 128
DE = 4
DEP = 16
DH = 300
DHP = 384
G = 128

NC, NS = 2, 16
NW = NC * NS
CHUNK = 64
CPW = 158
EPAD = NW * CPW * CHUNK
NCH = EPAD // CHUNK
XP = 10240
TROWS = EPAD + XP
NP = 10240
RPT = NP // NS
ZR = 128
EB = 1024
EBLK = EPAD // EB


def _sc_body(tab_hbm, sd_hbm, outx_hbm,
             aggx_sh, sd0, sd1, tbuf0, tbuf1,
             gsem0, gsem1, ssem0, ssem1):
    cid = lax.axis_index("c")
    sid = lax.axis_index("s")
    wid = cid * NS + sid
    sds = (sd0, sd1)
    tbufs = (tbuf0, tbuf1)
    gsems, ssems = (gsem0, gsem1), (ssem0, ssem1)

    zeros16 = jnp.zeros((16,), jnp.float32)

    @pl.loop(0, ZR)
    def _zero(i):
        for j in range(DF // 16):
            sl = pl.ds(j * 16, 16)
            tbuf0[i, sl] = zeros16
            tbuf1[i, sl] = zeros16

    def _ramp(ref, r, base):
        for j in range(2 * CHUNK // 16):
            ref[r, 0, pl.ds(j * 16, 16)] = lax.iota(jnp.int32, 16) + (base + j * 16)

    for k in range(RPT // ZR):
        r0 = sid * RPT + k * ZR
        _ramp(sd0, 0, r0)
        pltpu.sync_copy(tbuf0, aggx_sh.at[sd0.at[0, 0]])
    plsc.subcore_barrier()

    def _scat(b):
        return pltpu.make_async_copy(tbufs[b], aggx_sh.at[sds[b].at[1, 0]],
                                     ssems[b])

    for b in range(2):
        _ramp(sds[b], 1, 0)
        _scat(b).start(add=True)

    def _fire(c, b):
        _scat(b).wait()
        pltpu.sync_copy(sd_hbm.at[wid * CPW + c], sds[b])
        pltpu.async_copy(tab_hbm.at[sds[b].at[0, 0]], tbufs[b], gsems[b])

    def _drain_and_add(c, b):
        pltpu.make_async_copy(tab_hbm.at[sds[b].at[0, 0]], tbufs[b],
                              gsems[b]).wait()
        _scat(b).start(add=True)

    _fire(0, 0)

    @pl.loop(0, CPW // 2)
    def _edges(g):
        for b in range(2):
            c = g * 2 + b

            @pl.when(c + 1 < CPW)
            def _():
                _fire(c + 1, 1 - b)

            _drain_and_add(c, b)

    _scat(0).wait()
    _scat(1).wait()
    plsc.subcore_barrier()

    for k in range(RPT // ZR):
        r0 = sid * RPT + k * ZR
        o0 = cid * NP + r0
        _ramp(sd0, 0, r0)
        pltpu.sync_copy(aggx_sh.at[sd0.at[0, 0]], tbuf0)
        pltpu.sync_copy(tbuf0, outx_hbm.at[pl.ds(o0, ZR), :])


def _sc_agg(table, sd4):
    return pl.kernel(
        _sc_body,
        out_type=jax.ShapeDtypeStruct((NC * NP, DF), jnp.float32),
        mesh=plsc.VectorSubcoreMesh(core_axis_name="c", subcore_axis_name="s"),
        scratch_types=[
            pltpu.VMEM_SHARED((NP, DF), jnp.float32),
            pltpu.VMEM((2, 1, 2 * CHUNK), jnp.int32),
            pltpu.VMEM((2, 1, 2 * CHUNK), jnp.int32),
            pltpu.VMEM((2 * CHUNK, DF), jnp.float32),
            pltpu.VMEM((2 * CHUNK, DF), jnp.float32),
            pltpu.SemaphoreType.DMA,
            pltpu.SemaphoreType.DMA,
            pltpu.SemaphoreType.DMA,
            pltpu.SemaphoreType.DMA,
        ],
    )(table, sd4)


def _table_body(ea_ref, x_ref, we_ref, out_ref):
    pid = pl.program_id(0)
    mm = jnp.dot(ea_ref[...], we_ref[...], preferred_element_type=jnp.float32)
    out_ref[...] = jnp.where(pid < EBLK, mm, x_ref[...])


_table_call = pl.pallas_call(
    _table_body,
    grid=(EBLK + XP // EB,),
    in_specs=[
        pl.BlockSpec((EB, DEP), lambda i: (jnp.minimum(i, EBLK - 1), 0)),
        pl.BlockSpec((EB, DF), lambda i: (jnp.maximum(i - EBLK, 0), 0)),
        pl.BlockSpec((DEP, DF), lambda i: (0, 0)),
    ],
    out_specs=pl.BlockSpec((EB, DF), lambda i: (i, 0)),
    out_shape=jax.ShapeDtypeStruct((TROWS, DF), jnp.float32),
)


def _tc_body(paggx, x, batch, W_gnn, b_gnn, W_imp, b_imp,
             W_p1, b_p1, W_p2, b_p2, out_ref):
    f32 = jnp.float32
    t = paggx[0:N, :] + paggx[NP:NP + N, :] + x[...]

    imp_pre = jnp.dot(t, W_imp[...], preferred_element_type=f32) + b_imp[0, 0]
    imp = 1.0 / (1.0 + jnp.exp(-imp_pre))
    impT_pre = lax.dot_general(W_imp[...], t, (((0,), (1,)), ((), ())),
                               preferred_element_type=f32) + b_imp[0, 0]
    impT = 1.0 / (1.0 + jnp.exp(-impT_pre))

    h = jnp.maximum(jnp.dot(t, W_gnn[...], preferred_element_type=f32)
                    + b_gnn[...], 0.0)

    gid = lax.broadcasted_iota(jnp.int32, (G, N), 0)
    oh = jnp.broadcast_to(batch[...], (G, N)) == gid
    ohf = oh.astype(f32)

    impb = jnp.broadcast_to(impT[0:1, :], (G, N))
    m10 = jnp.max(jnp.where(oh, impb, -1.0), axis=1, keepdims=True) * 10.0
    node_m = lax.dot_general(ohf, m10, (((0,), (0,)), ((), ())),
                             preferred_element_type=f32)
    wgt = imp[:, 0:1] / node_m + 0.9

    hw = h * jnp.broadcast_to(wgt, (N, DHP))
    sums = jnp.dot(ohf, hw, preferred_element_type=f32)
    counts = jnp.sum(ohf, axis=1, keepdims=True)
    pooled = sums / jnp.maximum(counts, 1.0)

    hid = jnp.maximum(jnp.dot(pooled, W_p1[...], preferred_element_type=f32)
                      + b_p1[...], 0.0)
    out_ref[...] = jnp.dot(hid, W_p2[...], preferred_element_type=f32) + b_p2[...]


_tc_call = pl.pallas_call(
    _tc_body,
    out_shape=jax.ShapeDtypeStruct((G, DHP), jnp.float32),
)


def kernel(x, edge_index, edge_attr, batch, W_edge, W_gnn, b_gnn, W_imp, b_imp,
           W_p1, b_p1, W_p2, b_p2):
    src = edge_index[0]
    dst = edge_index[1]

    pad = EPAD - E
    x_pad = jnp.zeros((XP, DF), jnp.float32).at[:N].set(x)
    src_pad = jnp.concatenate([src, jnp.full((pad,), N, jnp.int32)])
    dst_pad = jnp.concatenate([dst, jnp.zeros((pad,), jnp.int32)])
    ea_pad = jnp.pad(edge_attr, ((0, pad), (0, DEP - DE)))
    W_edge_p = jnp.pad(W_edge, ((0, DEP - DE), (0, 0)))

    g2 = jnp.stack([(EPAD + src_pad).reshape(NCH, CHUNK),
                    jnp.arange(EPAD, dtype=jnp.int32).reshape(NCH, CHUNK)],
                   axis=1).reshape(NCH, 1, 2 * CHUNK)
    s2 = jnp.concatenate([dst_pad.reshape(NCH, 1, CHUNK)] * 2, axis=2)
    sd4 = jnp.stack([g2, s2], axis=1)

    table = _table_call(ea_pad, x_pad, W_edge_p)
    paggx = _sc_agg(table, sd4)

    W_gnn_p = jnp.pad(W_gnn, ((0, 0), (0, DHP - DH)))
    b_gnn_p = jnp.pad(b_gnn, (0, DHP - DH)).reshape(1, DHP)
    W_imp_p = jnp.broadcast_to(W_imp, (DF, 8))
    b_imp_p = b_imp.reshape(1, 1)
    W_p1_p = jnp.pad(W_p1, ((0, DHP - DH), (0, DHP - DH)))
    b_p1_p = jnp.pad(b_p1, (0, DHP - DH)).reshape(1, DHP)
    W_p2_p = jnp.pad(W_p2, ((0, DHP - DH), (0, DHP - DH)))
    b_p2_p = jnp.pad(b_p2, (0, DHP - DH)).reshape(1, DHP)
    batch2 = batch.reshape(1, N)

    z = _tc_call(paggx, x, batch2, W_gnn_p, b_gnn_p,
                 W_imp_p, b_imp_p, W_p1_p, b_p1_p, W_p2_p, b_p2_p)
    return z[:, :DH]
